# baseline (speedup 1.0000x reference)
import jax
import jax.numpy as jnp
from jax.experimental import pallas as pl

_T = 10
_O = 10
_NSD = 16
_EXP = 16
_GED = 32
_S = _T + _O

_BM = 512


def _body(ns_ref, ego_ref, w1n_ref, w1e_ref, b1_ref, w2_ref, b2_ref,
          w3_ref, b3_ref, out_ref):
    bm = ego_ref.shape[0]
    ns = ns_ref[...].astype(jnp.bfloat16)
    ego = ego_ref[...].astype(jnp.bfloat16)

    a1 = jnp.dot(ns, w1n_ref[...], preferred_element_type=jnp.float32)
    e1 = jnp.dot(ego, w1e_ref[...], preferred_element_type=jnp.float32)
    e1x = jnp.broadcast_to(e1[:, None, :], (bm, _S, 128)).reshape(bm * _S, 128)
    s = a1 + e1x + b1_ref[...]
    h1 = jnp.where(s > 0, s, jnp.exp(jnp.minimum(s, 0.0)) - 1.0)

    p2 = jnp.dot(h1.astype(jnp.bfloat16), w2_ref[...],
                 preferred_element_type=jnp.float32) + b2_ref[...]
    h2 = jnp.where(p2 > 0, p2, jnp.exp(jnp.minimum(p2, 0.0)) - 1.0)
    out_all = jnp.dot(h2.astype(jnp.bfloat16), w3_ref[...],
                      preferred_element_type=jnp.float32) + b3_ref[...]

    nanmask = jnp.isnan(s)
    m64 = jnp.logical_or(nanmask[:, :64], nanmask[:, 64:])
    feat = jnp.where(m64, -jnp.inf, out_all)

    f3 = feat.reshape(bm, _S, 64)
    tmax = jnp.max(f3[:, :_T, :], axis=1)
    omax = jnp.max(f3[:, _T:, :], axis=1)
    tglob = tmax[:, :_GED]
    tglob = jnp.where(jnp.isinf(tglob), jnp.float32(-2.0), tglob)
    oglob = omax[:, _GED:]
    out_ref[...] = jnp.concatenate([tglob, oglob], axis=1)


def kernel(ego_states, neighbor_states, tW1, tb1, tW2, tb2, tW3, tb3,
           oW1, ob1, oW2, ob2, oW3, ob3):
    B = ego_states.shape[0]
    ns_flat = neighbor_states.reshape(B * _S, _NSD)

    w1n = jnp.concatenate([tW1[:_NSD], oW1[:_NSD]],
                          axis=1).astype(jnp.bfloat16)
    w1e = jnp.concatenate([tW1[_NSD:], oW1[_NSD:]],
                          axis=1).astype(jnp.bfloat16)
    b1 = jnp.concatenate([tb1, ob1])[None, :]
    z2 = jnp.zeros_like(tW2)
    w2 = jnp.concatenate([
        jnp.concatenate([tW2, z2], axis=1),
        jnp.concatenate([z2, oW2], axis=1)],
        axis=0).astype(jnp.bfloat16)
    b2 = jnp.concatenate([tb2, ob2])[None, :]
    z3 = jnp.zeros_like(tW3)
    w3 = jnp.concatenate([
        jnp.concatenate([tW3, z3], axis=1),
        jnp.concatenate([z3, oW3], axis=1)],
        axis=0).astype(jnp.bfloat16)
    b3 = jnp.concatenate([tb3, ob3])[None, :]

    grid = (B // _BM,)
    return pl.pallas_call(
        _body,
        grid=grid,
        in_specs=[
            pl.BlockSpec((_BM * _S, _NSD), lambda i: (i, 0)),
            pl.BlockSpec((_BM, _EXP), lambda i: (i, 0)),
            pl.BlockSpec((_NSD, 128), lambda i: (0, 0)),
            pl.BlockSpec((_EXP, 128), lambda i: (0, 0)),
            pl.BlockSpec((1, 128), lambda i: (0, 0)),
            pl.BlockSpec((128, 64), lambda i: (0, 0)),
            pl.BlockSpec((1, 64), lambda i: (0, 0)),
            pl.BlockSpec((64, 64), lambda i: (0, 0)),
            pl.BlockSpec((1, 64), lambda i: (0, 0)),
        ],
        out_specs=pl.BlockSpec((_BM, 2 * _GED), lambda i: (i, 0)),
        out_shape=jax.ShapeDtypeStruct((B, 2 * _GED), jnp.float32),
    )(ns_flat, ego_states, w1n, w1e, b1, w2, b2, w3, b3)


# speedup vs baseline: 2.5335x; 2.5335x over previous
import jax
import jax.numpy as jnp
from jax.experimental import pallas as pl

_T = 10
_O = 10
_NSD = 16
_EXP = 16
_GED = 32
_S = _T + _O

_BM = 512


def _elu(x):
    return jnp.where(x > 0, x, jnp.exp(jnp.minimum(x, 0.0)) - 1.0)


def _body(x_ref, ego_ref, w1big_ref, w1e_ref, b1_ref, w2_ref, b2_ref,
          w3_ref, b3_ref, out_ref):
    x = x_ref[...].astype(jnp.bfloat16)
    ego = ego_ref[...].astype(jnp.bfloat16)

    x1 = jnp.dot(x, w1big_ref[...], preferred_element_type=jnp.float32)
    e1 = jnp.dot(ego, w1e_ref[...],
                 preferred_element_type=jnp.float32) + b1_ref[...]

    neg_inf = jnp.float32(-jnp.inf)
    tacc = None
    oacc = None
    for j in range(_S):
        s = x1[:, 128 * j:128 * (j + 1)] + e1
        h1 = _elu(s)
        p2 = jnp.dot(h1.astype(jnp.bfloat16), w2_ref[...],
                     preferred_element_type=jnp.float32) + b2_ref[...]
        h2 = _elu(p2)
        o = jnp.dot(h2.astype(jnp.bfloat16), w3_ref[...],
                    preferred_element_type=jnp.float32) + b3_ref[...]
        m = jnp.logical_or(jnp.isnan(s[:, :64]), jnp.isnan(s[:, 64:]))
        f = jnp.where(m, neg_inf, o)
        if j < _T:
            tacc = f if tacc is None else jnp.maximum(tacc, f)
        else:
            oacc = f if oacc is None else jnp.maximum(oacc, f)

    tglob = tacc[:, :_GED]
    tglob = jnp.where(jnp.isinf(tglob), jnp.float32(-2.0), tglob)
    oglob = oacc[:, _GED:]
    out_ref[...] = jnp.concatenate([tglob, oglob], axis=1)


def kernel(ego_states, neighbor_states, tW1, tb1, tW2, tb2, tW3, tb3,
           oW1, ob1, oW2, ob2, oW3, ob3):
    B = ego_states.shape[0]
    x = neighbor_states.reshape(B, _S * _NSD)

    w1n = jnp.concatenate([tW1[:_NSD], oW1[:_NSD]], axis=1)
    w1big = jnp.kron(jnp.eye(_S, dtype=tW1.dtype),
                     w1n).astype(jnp.bfloat16)
    w1e = jnp.concatenate([tW1[_NSD:], oW1[_NSD:]],
                          axis=1).astype(jnp.bfloat16)
    b1 = jnp.concatenate([tb1, ob1])[None, :]
    z2 = jnp.zeros_like(tW2)
    w2 = jnp.concatenate([
        jnp.concatenate([tW2, z2], axis=1),
        jnp.concatenate([z2, oW2], axis=1)],
        axis=0).astype(jnp.bfloat16)
    b2 = jnp.concatenate([tb2, ob2])[None, :]
    z3 = jnp.zeros_like(tW3)
    w3 = jnp.concatenate([
        jnp.concatenate([tW3, z3], axis=1),
        jnp.concatenate([z3, oW3], axis=1)],
        axis=0).astype(jnp.bfloat16)
    b3 = jnp.concatenate([tb3, ob3])[None, :]

    grid = (B // _BM,)
    return pl.pallas_call(
        _body,
        grid=grid,
        in_specs=[
            pl.BlockSpec((_BM, _S * _NSD), lambda i: (i, 0)),
            pl.BlockSpec((_BM, _EXP), lambda i: (i, 0)),
            pl.BlockSpec((_S * _NSD, _S * 128), lambda i: (0, 0)),
            pl.BlockSpec((_EXP, 128), lambda i: (0, 0)),
            pl.BlockSpec((1, 128), lambda i: (0, 0)),
            pl.BlockSpec((128, 64), lambda i: (0, 0)),
            pl.BlockSpec((1, 64), lambda i: (0, 0)),
            pl.BlockSpec((64, 64), lambda i: (0, 0)),
            pl.BlockSpec((1, 64), lambda i: (0, 0)),
        ],
        out_specs=pl.BlockSpec((_BM, 2 * _GED), lambda i: (i, 0)),
        out_shape=jax.ShapeDtypeStruct((B, 2 * _GED), jnp.float32),
    )(x, ego_states, w1big, w1e, b1, w2, b2, w3, b3)


# speedup vs baseline: 3.9259x; 1.5496x over previous
import jax
import jax.numpy as jnp
from jax.experimental import pallas as pl

_T = 10
_O = 10
_NSD = 16
_EXP = 16
_GED = 32
_S = _T + _O

_BM = 1024


def _elu(x):
    return jnp.where(x > 0, x, jnp.exp(x) - jnp.asarray(1.0, x.dtype))


def _body(x_ref, ego_ref, w1big_ref, w1et_ref, w1eo_ref, b1t_ref, b1o_ref,
          w2t_ref, w2o_ref, b2t_ref, b2o_ref, w3t_ref, w3o_ref,
          b3t_ref, b3o_ref, out_ref):
    x = x_ref[...].astype(jnp.bfloat16)
    ego = ego_ref[...].astype(jnp.bfloat16)

    x1 = jnp.dot(x, w1big_ref[...], preferred_element_type=jnp.float32)
    e1t = jnp.dot(ego, w1et_ref[...],
                  preferred_element_type=jnp.float32) + b1t_ref[...]
    e1o = jnp.dot(ego, w1eo_ref[...],
                  preferred_element_type=jnp.float32) + b1o_ref[...]

    neg_inf = jnp.float32(-jnp.inf)
    tacc = None
    oacc = None
    for j in range(_S):
        tm = j < _T
        e1 = e1t if tm else e1o
        s = x1[:, 64 * j:64 * (j + 1)] + e1
        h1 = _elu(s.astype(jnp.bfloat16))
        p2 = jnp.dot(h1, w2t_ref[...] if tm else w2o_ref[...],
                     preferred_element_type=jnp.float32)
        p2 = p2 + (b2t_ref[...] if tm else b2o_ref[...])
        h2 = _elu(p2.astype(jnp.bfloat16))
        o = jnp.dot(h2, w3t_ref[...] if tm else w3o_ref[...],
                    preferred_element_type=jnp.float32)
        o = o + (b3t_ref[...] if tm else b3o_ref[...])
        f = jnp.where(jnp.isnan(o), neg_inf, o)
        if tm:
            tacc = f if tacc is None else jnp.maximum(tacc, f)
        else:
            oacc = f if oacc is None else jnp.maximum(oacc, f)

    tglob = jnp.where(jnp.isinf(tacc), jnp.float32(-2.0), tacc)
    out_ref[...] = jnp.concatenate([tglob, oacc], axis=1)


def kernel(ego_states, neighbor_states, tW1, tb1, tW2, tb2, tW3, tb3,
           oW1, ob1, oW2, ob2, oW3, ob3):
    B = ego_states.shape[0]
    x = neighbor_states.reshape(B, _S * _NSD)

    eye_t = jnp.eye(_T, dtype=tW1.dtype)
    top = jnp.kron(eye_t, tW1[:_NSD])
    bot = jnp.kron(eye_t, oW1[:_NSD])
    zpad = jnp.zeros_like(top)
    w1big = jnp.concatenate([
        jnp.concatenate([top, zpad], axis=1),
        jnp.concatenate([zpad, bot], axis=1)],
        axis=0).astype(jnp.bfloat16)
    w1et = tW1[_NSD:].astype(jnp.bfloat16)
    w1eo = oW1[_NSD:].astype(jnp.bfloat16)
    b1t, b1o = tb1[None, :], ob1[None, :]
    w2t = tW2.astype(jnp.bfloat16)
    w2o = oW2.astype(jnp.bfloat16)
    b2t, b2o = tb2[None, :], ob2[None, :]
    w3t = tW3.astype(jnp.bfloat16)
    w3o = oW3.astype(jnp.bfloat16)
    b3t, b3o = tb3[None, :], ob3[None, :]

    grid = (B // _BM,)
    full = lambda i: (0, 0)
    return pl.pallas_call(
        _body,
        grid=grid,
        in_specs=[
            pl.BlockSpec((_BM, _S * _NSD), lambda i: (i, 0)),
            pl.BlockSpec((_BM, _EXP), lambda i: (i, 0)),
            pl.BlockSpec((_S * _NSD, _S * 64), full),
            pl.BlockSpec((_EXP, 64), full),
            pl.BlockSpec((_EXP, 64), full),
            pl.BlockSpec((1, 64), full),
            pl.BlockSpec((1, 64), full),
            pl.BlockSpec((64, 32), full),
            pl.BlockSpec((64, 32), full),
            pl.BlockSpec((1, 32), full),
            pl.BlockSpec((1, 32), full),
            pl.BlockSpec((32, 32), full),
            pl.BlockSpec((32, 32), full),
            pl.BlockSpec((1, 32), full),
            pl.BlockSpec((1, 32), full),
        ],
        out_specs=pl.BlockSpec((_BM, 2 * _GED), lambda i: (i, 0)),
        out_shape=jax.ShapeDtypeStruct((B, 2 * _GED), jnp.float32),
    )(x, ego_states, w1big, w1et, w1eo, b1t, b1o,
      w2t, w2o, b2t, b2o, w3t, w3o, b3t, b3o)
